# Initial kernel scaffold; baseline (speedup 1.0000x reference)
import jax
import jax.numpy as jnp
from jax.experimental import pallas as pl


def kernel(A, x):
    raise NotImplementedError("write your pallas kernel here")


# speedup vs baseline: 1.0310x; 1.0310x over previous
import functools

import jax
import jax.numpy as jnp
from jax.experimental import pallas as pl
from jax.experimental.pallas import tpu as pltpu

M = 16384
K = 16384
BM = 256


def _mv_block(x_ref, a_ref, o_ref):
    o_ref[...] = jax.lax.dot_general(
        a_ref[...], x_ref[...],
        dimension_numbers=(((1,), (0,)), ((), ())),
        preferred_element_type=jnp.float32,
    )


@jax.jit
def kernel(A, x):
    return pl.pallas_call(
        _mv_block,
        grid=(M // BM,),
        in_specs=[
            pl.BlockSpec((K,), lambda i: (0,)),
            pl.BlockSpec((BM, K), lambda i: (i, 0)),
        ],
        out_specs=pl.BlockSpec((BM,), lambda i: (i,)),
        out_shape=jax.ShapeDtypeStruct((M,), jnp.float32),
        compiler_params=pltpu.CompilerParams(
            dimension_semantics=("arbitrary",),
        ),
    )(x, A)
